# baseline (speedup 1.0000x reference)
import functools

import jax
import jax.numpy as jnp
from jax import lax
from jax.experimental import pallas as pl
from jax.experimental.pallas import tpu as pltpu
from jax.experimental.pallas import tpu_sc as plsc

D = 64
B = 4096
L = 200
N = B * L
NC, NS = 2, 16
NW = NC * NS
PER_W = N // NW
CHUNK = 128
NCHUNK = PER_W // CHUNK
NBUF = 4
ROUNDS = NCHUNK // NBUF


@functools.partial(
    pl.kernel,
    mesh=plsc.VectorSubcoreMesh(core_axis_name="c", subcore_axis_name="s"),
    out_type=jax.ShapeDtypeStruct((N, D), jnp.float32),
    scratch_types=[
        pltpu.VMEM((NCHUNK, CHUNK), jnp.int32),
        pltpu.VMEM((NBUF, CHUNK, D), jnp.float32),
    ]
    + [pltpu.SemaphoreType.DMA] * (2 * NBUF),
    compiler_params=pltpu.CompilerParams(use_tc_tiling_on_sc=False),
)
def _gather_kernel(idx_hbm, table_hbm, out_hbm, idx_v, bufs, *sems):
    gsem = sems[:NBUF]
    ssem = sems[NBUF:]
    wid = lax.axis_index("s") * NC + lax.axis_index("c")
    base = wid * PER_W

    pltpu.sync_copy(idx_hbm.at[wid], idx_v)

    for b in range(NBUF):
        pltpu.async_copy(table_hbm.at[idx_v.at[b]], bufs.at[b], gsem[b])

    def round_body(g, carry):
        for b in range(NBUF):
            c = g * NBUF + b
            row0 = base + c * CHUNK
            pltpu.make_async_copy(
                table_hbm.at[idx_v.at[c]], bufs.at[b], gsem[b]
            ).wait()
            pltpu.async_copy(bufs.at[b], out_hbm.at[pl.ds(row0, CHUNK)], ssem[b])
            pltpu.make_async_copy(
                bufs.at[b], out_hbm.at[pl.ds(row0, CHUNK)], ssem[b]
            ).wait()

            @pl.when(c + NBUF < NCHUNK)
            def _():
                pltpu.async_copy(
                    table_hbm.at[idx_v.at[c + NBUF]], bufs.at[b], gsem[b]
                )

        return carry

    lax.fori_loop(0, ROUNDS, round_body, 0)


def kernel(entity_pairs, table):
    idx = entity_pairs[:, :, 0].reshape(NW, NCHUNK, CHUNK)
    out = _gather_kernel(idx, table)
    return out.reshape(B, L, D)


# speedup vs baseline: 1.0019x; 1.0019x over previous
import functools

import jax
import jax.numpy as jnp
from jax import lax
from jax.experimental import pallas as pl
from jax.experimental.pallas import tpu as pltpu
from jax.experimental.pallas import tpu_sc as plsc

D = 64
B = 4096
L = 200
N = B * L
NC, NS = 2, 16
NW = NC * NS
PER_W = N // NW
CHUNK = 128
NCHUNK = PER_W // CHUNK
NBUF = 8
PF = 4
ROUNDS = NCHUNK // NBUF


@functools.partial(
    pl.kernel,
    mesh=plsc.VectorSubcoreMesh(core_axis_name="c", subcore_axis_name="s"),
    out_type=jax.ShapeDtypeStruct((N, D), jnp.float32),
    scratch_types=[
        pltpu.VMEM((NCHUNK, CHUNK), jnp.int32),
        pltpu.VMEM((NBUF, CHUNK, D), jnp.float32),
    ]
    + [pltpu.SemaphoreType.DMA] * (2 * NBUF),
    compiler_params=pltpu.CompilerParams(use_tc_tiling_on_sc=False),
)
def _gather_kernel(idx_hbm, table_hbm, out_hbm, idx_v, bufs, *sems):
    gsem = sems[:NBUF]
    ssem = sems[NBUF:]
    wid = lax.axis_index("s") * NC + lax.axis_index("c")
    base = wid * PER_W

    pltpu.sync_copy(idx_hbm.at[wid], idx_v)

    for b in range(PF):
        pltpu.async_copy(table_hbm.at[idx_v.at[b]], bufs.at[b], gsem[b])

    def round_body(g, carry):
        for b in range(NBUF):
            c = g * NBUF + b
            bp = (b + PF) % NBUF

            @pl.when(c + PF < NCHUNK)
            def _():
                @pl.when(c >= NBUF - PF)
                def _():
                    pltpu.make_async_copy(
                        bufs.at[bp], out_hbm.at[pl.ds(base, CHUNK)], ssem[bp]
                    ).wait()

                pltpu.async_copy(
                    table_hbm.at[idx_v.at[c + PF]], bufs.at[bp], gsem[bp]
                )

            pltpu.make_async_copy(
                table_hbm.at[idx_v.at[0]], bufs.at[b], gsem[b]
            ).wait()
            pltpu.async_copy(
                bufs.at[b], out_hbm.at[pl.ds(base + c * CHUNK, CHUNK)], ssem[b]
            )

        return carry

    lax.fori_loop(0, ROUNDS, round_body, 0)

    for b in range(NBUF):
        pltpu.make_async_copy(
            bufs.at[b], out_hbm.at[pl.ds(base, CHUNK)], ssem[b]
        ).wait()


def kernel(entity_pairs, table):
    idx = entity_pairs[:, :, 0].reshape(NW, NCHUNK, CHUNK)
    out = _gather_kernel(idx, table)
    return out.reshape(B, L, D)
